# Initial kernel scaffold; baseline (speedup 1.0000x reference)
import jax
import jax.numpy as jnp
from jax.experimental import pallas as pl


def kernel(output, target):
    raise NotImplementedError("write your pallas kernel here")


# speedup vs baseline: 1.0154x; 1.0154x over previous
import functools

import jax
import jax.numpy as jnp
from jax import lax
from jax.experimental import pallas as pl
from jax.experimental.pallas import tpu as pltpu
from jax.experimental.pallas import tpu_sc as plsc

NC = 2
NS = 16
L = 16
NW = NC * NS

B = 128
V = 100000
ROWS_PER_W = B // NW
CH = 20000
NCH = V // CH
VECS = CH // L
NEG_INF = float("-inf")

_GATHER_DNUMS = lax.GatherDimensionNumbers(
    offset_dims=(), collapsed_slice_dims=(0,), start_index_map=(0,)
)


def _perm(x, idx):
    return lax.gather(
        x,
        idx[:, None],
        dimension_numbers=_GATHER_DNUMS,
        slice_sizes=(1,),
        mode=lax.GatherScatterMode.PROMISE_IN_BOUNDS,
    )


def _sc_body(out_hbm, tgt_hbm, part_hbm, bufa, bufb, tgt_v, cnt_v, sem0, sem1):
    c = lax.axis_index("c")
    s = lax.axis_index("s")
    wid = s * NC + c
    row0 = wid * ROWS_PER_W

    total = ROWS_PER_W * NCH
    sems = (sem0, sem1)
    bufs = (bufa, bufb)

    def src(t):
        k, ch = divmod(t, NCH)
        return out_hbm.at[pl.ds((row0 + k) * V + ch * CH, CH)]

    pend = [None, None]
    pend[0] = pltpu.async_copy(src(0), bufs[0], sems[0])
    pltpu.sync_copy(tgt_hbm, tgt_v)
    base = jnp.minimum(8 * (wid // 2), B - L)
    tvec = tgt_v[pl.ds(base, L)]
    lane0 = 4 * wid - base
    lane_iota = lax.iota(jnp.int32, L)

    count = jnp.zeros((L,), jnp.float32)
    amax = jnp.full((L,), NEG_INF, jnp.float32)
    aidx = jnp.zeros((L,), jnp.int32)

    for t in range(total):
        k, ch = divmod(t, NCH)
        if ch == 0:
            amax = jnp.full((L,), NEG_INF, jnp.float32)
            aidx = jnp.zeros((L,), jnp.int32)
        if t + 1 < total:
            pend[(t + 1) % 2] = pltpu.async_copy(
                src(t + 1), bufs[(t + 1) % 2], sems[(t + 1) % 2]
            )
        pend[t % 2].wait()

        buf = bufs[t % 2]
        vbase = ch * VECS

        def vstep(j, carry, buf=buf, vbase=vbase):
            am, ai = carry
            v = buf[pl.ds(j * L, L)]
            gt = v > am
            am = jnp.where(gt, v, am)
            ai = jnp.where(gt, jnp.broadcast_to(vbase + j, (L,)).astype(jnp.int32), ai)
            return am, ai

        amax, aidx = lax.fori_loop(0, VECS, vstep, (amax, aidx), unroll=8)

        if ch == NCH - 1:
            mv = amax
            mi = aidx * L + lane_iota
            for shift in (1, 2, 4, 8):
                pv = _perm(mv, lane_iota ^ shift)
                pi = _perm(mi, lane_iota ^ shift)
                take = (pv > mv) | ((pv == mv) & (pi < mi))
                mv = jnp.where(take, pv, mv)
                mi = jnp.where(take, pi, mi)
            ok = (tvec == mi) & (lane_iota == lane0 + k)
            count = count + jnp.where(ok, 1.0, 0.0).astype(jnp.float32)

    cnt_v[...] = count
    pltpu.sync_copy(cnt_v, part_hbm.at[wid])


@functools.cache
def _sc_partial():
    return functools.partial(
        pl.kernel,
        out_type=jax.ShapeDtypeStruct((NW, L), jnp.float32),
        mesh=plsc.VectorSubcoreMesh(
            core_axis_name="c", subcore_axis_name="s", num_cores=NC, num_subcores=NS
        ),
        scratch_types=[
            pltpu.VMEM((CH,), jnp.float32),
            pltpu.VMEM((CH,), jnp.float32),
            pltpu.VMEM((B,), jnp.int32),
            pltpu.VMEM((L,), jnp.float32),
            pltpu.SemaphoreType.DMA,
            pltpu.SemaphoreType.DMA,
        ],
    )(_sc_body)


def _tc_finalize_body(p_ref, o_ref):
    o_ref[0] = jnp.sum(p_ref[...]) * (100.0 / B)


_tc_finalize = pl.pallas_call(
    _tc_finalize_body,
    out_shape=jax.ShapeDtypeStruct((1,), jnp.float32),
    in_specs=[pl.BlockSpec(memory_space=pltpu.VMEM)],
    out_specs=pl.BlockSpec(memory_space=pltpu.SMEM),
)


@jax.jit
def kernel(output, target):
    partial = _sc_partial()(output.reshape(-1), target)
    return _tc_finalize(partial)


# speedup vs baseline: 1.3463x; 1.3259x over previous
import functools

import jax
import jax.numpy as jnp
from jax import lax
from jax.experimental import pallas as pl
from jax.experimental.pallas import tpu as pltpu
from jax.experimental.pallas import tpu_sc as plsc

NC = 2
NS = 16
L = 16
NW = NC * NS

B = 128
V = 100000
RPG = 8
G = B // RPG

NSLAB = 4
SLAB_W = 24960
MW = 12416
CHUNK_TILES = (39, 39, 19)
CWMAX = 39 * 128
NEG_INF = float("-inf")

_GATHER_DNUMS = lax.GatherDimensionNumbers(
    offset_dims=(), collapsed_slice_dims=(0,), start_index_map=(0,)
)


def _perm(x, idx):
    return lax.gather(
        x,
        idx[:, None],
        dimension_numbers=_GATHER_DNUMS,
        slice_sizes=(1,),
        mode=lax.GatherScatterMode.PROMISE_IN_BOUNDS,
    )


def _make_sc_body(slab_off, ragged_w):
    chunk_ws = [t * 128 for t in CHUNK_TILES]
    n_chunks = len(chunk_ws)
    chunk_offs = [sum(chunk_ws[:i]) for i in range(n_chunks)]

    def body(
        out_hbm, mx_hbm, ix_hbm, bufa, bufb, bufc, buft, mx_v, ix_v,
        sem0, sem1, sem2, semt,
    ):
        c = lax.axis_index("c")
        s = lax.axis_index("s")
        wid = s * NC + c
        g = wid % G
        h = wid // G
        r0 = g * RPG
        lbase = h * MW

        bufs = (bufa, bufb, bufc)
        sems = (sem0, sem1, sem2)

        def src(t):
            return out_hbm.at[pl.ds(r0, RPG), pl.ds(lbase + chunk_offs[t], chunk_ws[t])]

        pend = [pltpu.async_copy(src(t), bufs[t], sems[t]) for t in range(n_chunks)]
        tail_cp = pltpu.async_copy(
            out_hbm.at[pl.ds(r0, RPG), pl.ds(2 * MW, ragged_w)],
            buft,
            semt,
        )

        lane_iota = lax.iota(jnp.int32, L)
        negv = jnp.full((L,), NEG_INF, jnp.float32)
        zeroi = jnp.zeros((L,), jnp.int32)
        ams = [negv] * RPG
        ais = [zeroi] * RPG

        for t in range(n_chunks):
            pend[t].wait()
            buf = bufs[t]
            vb = (slab_off + chunk_offs[t]) // L + h * (MW // L)

            def step(j, carry, buf=buf, vb=vb):
                accs = list(carry)
                idxv = jnp.broadcast_to(vb + j, (L,)).astype(jnp.int32)
                for i in range(RPG):
                    am = accs[2 * i]
                    ai = accs[2 * i + 1]
                    v = buf[i, pl.ds(j * L, L)]
                    gt = v > am
                    accs[2 * i] = jnp.where(gt, v, am)
                    accs[2 * i + 1] = jnp.where(gt, idxv, ai)
                return tuple(accs)

            carry = tuple(x for pair in zip(ams, ais) for x in pair)
            carry = lax.fori_loop(0, chunk_ws[t] // L, step, carry, unroll=2)
            ams = list(carry[0::2])
            ais = list(carry[1::2])

        tail_cp.wait()
        hcap = jnp.broadcast_to(2 * h - 1, (L,)).astype(jnp.float32) * jnp.float32(
            float("inf")
        )
        tail_vb = (slab_off + 2 * MW) // L
        for j in range(ragged_w // L):
            idxv = jnp.broadcast_to(jnp.int32(tail_vb + j), (L,))
            for i in range(RPG):
                v = jnp.minimum(buft[i, pl.ds(j * L, L)], hcap)
                gt = v > ams[i]
                ams[i] = jnp.where(gt, v, ams[i])
                ais[i] = jnp.where(gt, idxv, ais[i])

        mxv = negv
        ixv = zeroi
        for i in range(RPG):
            mv = ams[i]
            mi = ais[i] * L + lane_iota
            for shift in (1, 2, 4, 8):
                pv = _perm(mv, lane_iota ^ shift)
                pi = _perm(mi, lane_iota ^ shift)
                take = (pv > mv) | ((pv == mv) & (pi < mi))
                mv = jnp.where(take, pv, mv)
                mi = jnp.where(take, pi, mi)
            sel = lane_iota == i
            mxv = jnp.where(sel, mv, mxv)
            ixv = jnp.where(sel, mi, ixv)

        mx_v[...] = mxv
        ix_v[...] = ixv
        pltpu.sync_copy(mx_v, mx_hbm.at[wid])
        pltpu.sync_copy(ix_v, ix_hbm.at[wid])

    return body


@functools.cache
def _sc_slab(slab_off, slab_w):
    ragged_w = slab_w - 2 * MW
    return functools.partial(
        pl.kernel,
        out_type=(
            jax.ShapeDtypeStruct((NW, L), jnp.float32),
            jax.ShapeDtypeStruct((NW, L), jnp.int32),
        ),
        mesh=plsc.VectorSubcoreMesh(
            core_axis_name="c", subcore_axis_name="s", num_cores=NC, num_subcores=NS
        ),
        scratch_types=[
            pltpu.VMEM((RPG, CHUNK_TILES[0] * 128), jnp.float32),
            pltpu.VMEM((RPG, CHUNK_TILES[1] * 128), jnp.float32),
            pltpu.VMEM((RPG, CHUNK_TILES[2] * 128), jnp.float32),
            pltpu.VMEM((RPG, ragged_w), jnp.float32),
            pltpu.VMEM((L,), jnp.float32),
            pltpu.VMEM((L,), jnp.int32),
            pltpu.SemaphoreType.DMA,
            pltpu.SemaphoreType.DMA,
            pltpu.SemaphoreType.DMA,
            pltpu.SemaphoreType.DMA,
        ],
    )(_make_sc_body(slab_off, ragged_w))


def _tc_finalize_body(*refs):
    mx_refs = refs[:NSLAB]
    ix_refs = refs[NSLAB : 2 * NSLAB]
    t_ref = refs[2 * NSLAB]
    o_ref = refs[2 * NSLAB + 1]
    best_m = jnp.full((G, L), NEG_INF, jnp.float32)
    best_i = jnp.zeros((G, L), jnp.int32)
    for k in range(NSLAB):
        for h in range(2):
            m = mx_refs[k][h * G : (h + 1) * G, :]
            i = ix_refs[k][h * G : (h + 1) * G, :]
            take = (m > best_m) | ((m == best_m) & (i < best_i))
            best_m = jnp.where(take, m, best_m)
            best_i = jnp.where(take, i, best_i)
    ok = (best_i == t_ref[...]).astype(jnp.float32)
    o_ref[0] = jnp.sum(ok) * (100.0 / B)


_tc_finalize = pl.pallas_call(
    _tc_finalize_body,
    out_shape=jax.ShapeDtypeStruct((1,), jnp.float32),
    in_specs=[pl.BlockSpec(memory_space=pltpu.VMEM)] * (2 * NSLAB + 1),
    out_specs=pl.BlockSpec(memory_space=pltpu.SMEM),
)


@jax.jit
def kernel(output, target):
    mxs = []
    ixs = []
    for k in range(NSLAB):
        off = k * SLAB_W
        w = SLAB_W if k + 1 < NSLAB else V - off
        mx, ix = _sc_slab(off, w)(output[:, off : off + w])
        mxs.append(mx)
        ixs.append(ix)
    tpad = jnp.pad(
        target.reshape(G, RPG), ((0, 0), (0, L - RPG)), constant_values=-1
    )
    return _tc_finalize(*mxs, *ixs, tpad)


# speedup vs baseline: 1.5555x; 1.1554x over previous
import functools

import jax
import jax.numpy as jnp
from jax import lax
from jax.experimental import pallas as pl
from jax.experimental.pallas import tpu as pltpu
from jax.experimental.pallas import tpu_sc as plsc

NC = 2
NS = 16
L = 16
NW = NC * NS

B = 128
V = 100000
RPG = 8
G = B // RPG

MW = 19968
SCW = 2 * MW
CW = 4992
NCHK = MW // CW

TBW = 1024
TC_OFF = SCW // TBW
TC_STEPS = (V + TBW - 1) // TBW - TC_OFF

NEG_INF = float("-inf")
BIG_I = 2**30

_GATHER_DNUMS = lax.GatherDimensionNumbers(
    offset_dims=(), collapsed_slice_dims=(0,), start_index_map=(0,)
)


def _perm(x, idx):
    return lax.gather(
        x,
        idx[:, None],
        dimension_numbers=_GATHER_DNUMS,
        slice_sizes=(1,),
        mode=lax.GatherScatterMode.PROMISE_IN_BOUNDS,
    )


def _sc_body(out_hbm, mx_hbm, ix_hbm, bufa, bufb, mx_v, ix_v, sem0, sem1):
    c = lax.axis_index("c")
    s = lax.axis_index("s")
    wid = s * NC + c
    g = wid % G
    h = wid // G
    r0 = g * RPG
    cbase = h * MW

    bufs = (bufa, bufb)
    sems = (sem0, sem1)

    def src(t):
        return out_hbm.at[pl.ds(r0, RPG), pl.ds(cbase + t * CW, CW)]

    pend = [None, None]
    pend[0] = pltpu.async_copy(src(0), bufs[0], sems[0])

    lane_iota = lax.iota(jnp.int32, L)
    negv = jnp.full((L,), NEG_INF, jnp.float32)
    zeroi = jnp.zeros((L,), jnp.int32)
    ams = [negv] * RPG
    ais = [zeroi] * RPG

    for t in range(NCHK):
        if t + 1 < NCHK:
            pend[(t + 1) % 2] = pltpu.async_copy(
                src(t + 1), bufs[(t + 1) % 2], sems[(t + 1) % 2]
            )
        pend[t % 2].wait()
        buf = bufs[t % 2]
        vb = (cbase + t * CW) // L

        def step(j, carry, buf=buf, vb=vb):
            accs = list(carry)
            idxv = jnp.broadcast_to(vb + j, (L,)).astype(jnp.int32)
            for i in range(RPG):
                am = accs[2 * i]
                ai = accs[2 * i + 1]
                v = buf[i, pl.ds(j * L, L)]
                gt = v > am
                accs[2 * i] = jnp.where(gt, v, am)
                accs[2 * i + 1] = jnp.where(gt, idxv, ai)
            return tuple(accs)

        carry = tuple(x for pair in zip(ams, ais) for x in pair)
        carry = lax.fori_loop(0, CW // L, step, carry, unroll=2)
        ams = list(carry[0::2])
        ais = list(carry[1::2])

    mxv = negv
    ixv = zeroi
    for i in range(RPG):
        mv = ams[i]
        mi = ais[i] * L + lane_iota
        for shift in (1, 2, 4, 8):
            pv = _perm(mv, lane_iota ^ shift)
            pi = _perm(mi, lane_iota ^ shift)
            take = (pv > mv) | ((pv == mv) & (pi < mi))
            mv = jnp.where(take, pv, mv)
            mi = jnp.where(take, pi, mi)
        sel = lane_iota == i
        mxv = jnp.where(sel, mv, mxv)
        ixv = jnp.where(sel, mi, ixv)

    mx_v[...] = mxv
    ix_v[...] = ixv
    pltpu.sync_copy(mx_v, mx_hbm.at[wid])
    pltpu.sync_copy(ix_v, ix_hbm.at[wid])


@functools.cache
def _sc_rowmax():
    return functools.partial(
        pl.kernel,
        out_type=(
            jax.ShapeDtypeStruct((NW, L), jnp.float32),
            jax.ShapeDtypeStruct((NW, L), jnp.int32),
        ),
        mesh=plsc.VectorSubcoreMesh(
            core_axis_name="c", subcore_axis_name="s", num_cores=NC, num_subcores=NS
        ),
        scratch_types=[
            pltpu.VMEM((RPG, CW), jnp.float32),
            pltpu.VMEM((RPG, CW), jnp.float32),
            pltpu.VMEM((L,), jnp.float32),
            pltpu.VMEM((L,), jnp.int32),
            pltpu.SemaphoreType.DMA,
            pltpu.SemaphoreType.DMA,
        ],
    )(_sc_body)


def _tc_rowmax_body(x_ref, mx_ref, ix_ref, am_ref, ai_ref):
    i = pl.program_id(0)

    @pl.when(i == 0)
    def _init():
        am_ref[...] = jnp.full((B, 128), NEG_INF, jnp.float32)
        ai_ref[...] = jnp.zeros((B, 128), jnp.int32)

    col0 = (TC_OFF + i) * TBW
    lane = jax.lax.broadcasted_iota(jnp.int32, (B, 128), 1)
    am = am_ref[...]
    ai = ai_ref[...]
    for j in range(TBW // 128):
        col = col0 + j * 128 + lane
        v = x_ref[:, j * 128 : (j + 1) * 128]
        v = jnp.where(col < V, v, NEG_INF)
        gt = v > am
        am = jnp.where(gt, v, am)
        ai = jnp.where(gt, col, ai)
    am_ref[...] = am
    ai_ref[...] = ai

    @pl.when(i == TC_STEPS - 1)
    def _final():
        m = jnp.max(am, axis=1, keepdims=True)
        cand = jnp.where(am == m, ai, BIG_I)
        mx_ref[...] = m
        ix_ref[...] = jnp.min(cand, axis=1, keepdims=True)


@functools.cache
def _tc_rowmax():
    return pl.pallas_call(
        _tc_rowmax_body,
        grid=(TC_STEPS,),
        in_specs=[pl.BlockSpec((B, TBW), lambda i: (0, TC_OFF + i))],
        out_specs=[
            pl.BlockSpec((B, 1), lambda i: (0, 0)),
            pl.BlockSpec((B, 1), lambda i: (0, 0)),
        ],
        out_shape=(
            jax.ShapeDtypeStruct((B, 1), jnp.float32),
            jax.ShapeDtypeStruct((B, 1), jnp.int32),
        ),
        scratch_shapes=[
            pltpu.VMEM((B, 128), jnp.float32),
            pltpu.VMEM((B, 128), jnp.int32),
        ],
    )


def _finalize_body(mx_ref, ix_ref, tm_ref, ti_ref, t_ref, o_ref):
    best_m = jnp.full((G, L), NEG_INF, jnp.float32)
    best_i = jnp.zeros((G, L), jnp.int32)
    for cm, ci in (
        (mx_ref[0:G, :], ix_ref[0:G, :]),
        (mx_ref[G : 2 * G, :], ix_ref[G : 2 * G, :]),
        (tm_ref[...], ti_ref[...]),
    ):
        take = (cm > best_m) | ((cm == best_m) & (ci < best_i))
        best_m = jnp.where(take, cm, best_m)
        best_i = jnp.where(take, ci, best_i)
    ok = (best_i == t_ref[...]).astype(jnp.float32)
    o_ref[0] = jnp.sum(ok) * (100.0 / B)


_finalize = pl.pallas_call(
    _finalize_body,
    out_shape=jax.ShapeDtypeStruct((1,), jnp.float32),
    in_specs=[pl.BlockSpec(memory_space=pltpu.VMEM)] * 5,
    out_specs=pl.BlockSpec(memory_space=pltpu.SMEM),
)


@jax.jit
def kernel(output, target):
    mx, ix = _sc_rowmax()(output)
    tm, ti = _tc_rowmax()(output)
    tm2 = jnp.pad(
        tm.reshape(G, RPG), ((0, 0), (0, L - RPG)), constant_values=NEG_INF
    )
    ti2 = jnp.pad(ti.reshape(G, RPG), ((0, 0), (0, L - RPG)))
    tpad = jnp.pad(
        target.reshape(G, RPG), ((0, 0), (0, L - RPG)), constant_values=-1
    )
    return _finalize(mx, ix, tm2, ti2, tpad)


# speedup vs baseline: 1.6029x; 1.0305x over previous
import functools

import jax
import jax.numpy as jnp
from jax import lax
from jax.experimental import pallas as pl
from jax.experimental.pallas import tpu as pltpu
from jax.experimental.pallas import tpu_sc as plsc

NC = 2
NS = 16
L = 16
NW = NC * NS

B = 128
V = 100000
RPG = 8
G = B // RPG

MW = 19968
SCW = 2 * MW
CW = 4992
NCHK = MW // CW

TBW = 1024
TC_OFF = SCW // TBW
TC_STEPS = (V + TBW - 1) // TBW - TC_OFF

NEG_INF = float("-inf")
BIG_I = 2**30

_GATHER_DNUMS = lax.GatherDimensionNumbers(
    offset_dims=(), collapsed_slice_dims=(0,), start_index_map=(0,)
)


def _perm(x, idx):
    return lax.gather(
        x,
        idx[:, None],
        dimension_numbers=_GATHER_DNUMS,
        slice_sizes=(1,),
        mode=lax.GatherScatterMode.PROMISE_IN_BOUNDS,
    )


def _sc_body(out_hbm, mx_hbm, ix_hbm, bufa, bufb, mx_v, ix_v, sem0, sem1):
    c = lax.axis_index("c")
    s = lax.axis_index("s")
    wid = s * NC + c
    g = wid % G
    h = wid // G
    r0 = g * RPG
    cbase = h * MW

    bufs = (bufa, bufb)
    sems = (sem0, sem1)

    def src(t):
        return out_hbm.at[pl.ds(r0, RPG), pl.ds(cbase + t * CW, CW)]

    pend = [None, None]
    pend[0] = pltpu.async_copy(src(0), bufs[0], sems[0])

    lane_iota = lax.iota(jnp.int32, L)
    negv = jnp.full((L,), NEG_INF, jnp.float32)
    zeroi = jnp.zeros((L,), jnp.int32)
    ams = [negv] * RPG
    ais = [zeroi] * RPG

    for t in range(NCHK):
        if t + 1 < NCHK:
            pend[(t + 1) % 2] = pltpu.async_copy(
                src(t + 1), bufs[(t + 1) % 2], sems[(t + 1) % 2]
            )
        pend[t % 2].wait()
        buf = bufs[t % 2]
        vb = (cbase + t * CW) // L

        def step(j, carry, buf=buf, vb=vb):
            accs = list(carry)
            idxv = jnp.broadcast_to(vb + j, (L,)).astype(jnp.int32)
            for i in range(RPG):
                am = accs[2 * i]
                ai = accs[2 * i + 1]
                v = buf[i, pl.ds(j * L, L)]
                gt = v > am
                accs[2 * i] = jnp.where(gt, v, am)
                accs[2 * i + 1] = jnp.where(gt, idxv, ai)
            return tuple(accs)

        carry = tuple(x for pair in zip(ams, ais) for x in pair)
        carry = lax.fori_loop(0, CW // L, step, carry, unroll=2)
        ams = list(carry[0::2])
        ais = list(carry[1::2])

    mxv = negv
    ixv = zeroi
    for i in range(RPG):
        mv = ams[i]
        mi = ais[i] * L + lane_iota
        for shift in (1, 2, 4, 8):
            pv = _perm(mv, lane_iota ^ shift)
            pi = _perm(mi, lane_iota ^ shift)
            take = (pv > mv) | ((pv == mv) & (pi < mi))
            mv = jnp.where(take, pv, mv)
            mi = jnp.where(take, pi, mi)
        sel = lane_iota == i
        mxv = jnp.where(sel, mv, mxv)
        ixv = jnp.where(sel, mi, ixv)

    mx_v[...] = mxv
    ix_v[...] = ixv
    pltpu.sync_copy(mx_v, mx_hbm.at[wid])
    pltpu.sync_copy(ix_v, ix_hbm.at[wid])


@functools.cache
def _sc_rowmax():
    return functools.partial(
        pl.kernel,
        out_type=(
            jax.ShapeDtypeStruct((NW, L), jnp.float32),
            jax.ShapeDtypeStruct((NW, L), jnp.int32),
        ),
        mesh=plsc.VectorSubcoreMesh(
            core_axis_name="c", subcore_axis_name="s", num_cores=NC, num_subcores=NS
        ),
        scratch_types=[
            pltpu.VMEM((RPG, CW), jnp.float32),
            pltpu.VMEM((RPG, CW), jnp.float32),
            pltpu.VMEM((L,), jnp.float32),
            pltpu.VMEM((L,), jnp.int32),
            pltpu.SemaphoreType.DMA,
            pltpu.SemaphoreType.DMA,
        ],
    )(_sc_body)


def _tc_rowmax_body(x_ref, mx_ref, ix_ref, am_ref, ai_ref):
    i = pl.program_id(0)

    @pl.when(i == 0)
    def _init():
        am_ref[...] = jnp.full((B, 128), NEG_INF, jnp.float32)
        ai_ref[...] = jnp.zeros((B, 128), jnp.int32)

    am = am_ref[...]
    ai = ai_ref[...]

    def fold(am, ai, masked):
        for j in range(TBW // 128):
            v = x_ref[:, j * 128 : (j + 1) * 128]
            if masked:
                col = (TC_OFF + i) * TBW + j * 128 + jax.lax.broadcasted_iota(
                    jnp.int32, (B, 128), 1
                )
                v = jnp.where(col < V, v, NEG_INF)
            step_id = jnp.broadcast_to(i * (TBW // 128) + j, (B, 128)).astype(
                jnp.int32
            )
            gt = v > am
            am = jnp.maximum(am, v)
            ai = jnp.where(gt, step_id, ai)
        return am, ai

    @pl.when(i < TC_STEPS - 1)
    def _hot():
        a, b = fold(am, ai, masked=False)
        am_ref[...] = a
        ai_ref[...] = b

    @pl.when(i == TC_STEPS - 1)
    def _last():
        a, b = fold(am, ai, masked=True)
        lane = jax.lax.broadcasted_iota(jnp.int32, (B, 128), 1)
        cols = b * 128 + lane + SCW
        m = jnp.max(a, axis=1, keepdims=True)
        cand = jnp.where(a == m, cols, BIG_I)
        mx_ref[...] = m
        ix_ref[...] = jnp.min(cand, axis=1, keepdims=True)


@functools.cache
def _tc_rowmax():
    return pl.pallas_call(
        _tc_rowmax_body,
        grid=(TC_STEPS,),
        in_specs=[pl.BlockSpec((B, TBW), lambda i: (0, TC_OFF + i))],
        out_specs=[
            pl.BlockSpec((B, 1), lambda i: (0, 0)),
            pl.BlockSpec((B, 1), lambda i: (0, 0)),
        ],
        out_shape=(
            jax.ShapeDtypeStruct((B, 1), jnp.float32),
            jax.ShapeDtypeStruct((B, 1), jnp.int32),
        ),
        scratch_shapes=[
            pltpu.VMEM((B, 128), jnp.float32),
            pltpu.VMEM((B, 128), jnp.int32),
        ],
    )


def _finalize_body(mx_ref, ix_ref, tm_ref, ti_ref, t_ref, o_ref):
    tm = jnp.concatenate(
        [tm_ref[...].reshape(G, RPG), jnp.full((G, L - RPG), NEG_INF, jnp.float32)],
        axis=1,
    )
    ti = jnp.concatenate(
        [ti_ref[...].reshape(G, RPG), jnp.zeros((G, L - RPG), jnp.int32)], axis=1
    )
    best_m = jnp.full((G, L), NEG_INF, jnp.float32)
    best_i = jnp.zeros((G, L), jnp.int32)
    for cm, ci in (
        (mx_ref[0:G, :], ix_ref[0:G, :]),
        (mx_ref[G : 2 * G, :], ix_ref[G : 2 * G, :]),
        (tm, ti),
    ):
        take = (cm > best_m) | ((cm == best_m) & (ci < best_i))
        best_m = jnp.where(take, cm, best_m)
        best_i = jnp.where(take, ci, best_i)
    ok = (best_i == t_ref[...]).astype(jnp.float32)
    o_ref[0] = jnp.sum(ok) * (100.0 / B)


_finalize = pl.pallas_call(
    _finalize_body,
    out_shape=jax.ShapeDtypeStruct((1,), jnp.float32),
    in_specs=[pl.BlockSpec(memory_space=pltpu.VMEM)] * 5,
    out_specs=pl.BlockSpec(memory_space=pltpu.SMEM),
)


@jax.jit
def kernel(output, target):
    mx, ix = _sc_rowmax()(output)
    tm, ti = _tc_rowmax()(output)
    tpad = jnp.pad(
        target.reshape(G, RPG), ((0, 0), (0, L - RPG)), constant_values=-1
    )
    return _finalize(mx, ix, tm, ti, tpad)


# speedup vs baseline: 1.6304x; 1.0172x over previous
import functools

import jax
import jax.numpy as jnp
from jax import lax
from jax.experimental import pallas as pl
from jax.experimental.pallas import tpu as pltpu
from jax.experimental.pallas import tpu_sc as plsc

NC = 2
NS = 16
L = 16
NW = NC * NS

B = 128
V = 100000
RPG = 8
G = B // RPG

MW = 19968
SCW = 2 * MW
CW = 4992
NCHK = MW // CW

TBW = 1024
TC_OFF = SCW // TBW
TC_STEPS = (V + TBW - 1) // TBW - TC_OFF

NEG_INF = float("-inf")
BIG_I = 2**30

_GATHER_DNUMS = lax.GatherDimensionNumbers(
    offset_dims=(), collapsed_slice_dims=(0,), start_index_map=(0,)
)


def _perm(x, idx):
    return lax.gather(
        x,
        idx[:, None],
        dimension_numbers=_GATHER_DNUMS,
        slice_sizes=(1,),
        mode=lax.GatherScatterMode.PROMISE_IN_BOUNDS,
    )


def _sc_body(out_hbm, mx_hbm, ix_hbm, bufa, bufb, mx_v, ix_v, sem0, sem1):
    c = lax.axis_index("c")
    s = lax.axis_index("s")
    wid = s * NC + c
    g = wid % G
    h = wid // G
    r0 = g * RPG
    cbase = h * MW

    bufs = (bufa, bufb)
    sems = (sem0, sem1)

    def src(t):
        return out_hbm.at[pl.ds(r0, RPG), pl.ds(cbase + t * CW, CW)]

    pend = [None, None]
    pend[0] = pltpu.async_copy(src(0), bufs[0], sems[0])

    lane_iota = lax.iota(jnp.int32, L)
    negv = jnp.full((L,), NEG_INF, jnp.float32)
    zeroi = jnp.zeros((L,), jnp.int32)
    ams = [negv] * RPG
    ais = [zeroi] * RPG

    for t in range(NCHK):
        if t + 1 < NCHK:
            pend[(t + 1) % 2] = pltpu.async_copy(
                src(t + 1), bufs[(t + 1) % 2], sems[(t + 1) % 2]
            )
        pend[t % 2].wait()
        buf = bufs[t % 2]
        vb = (cbase + t * CW) // L

        def step(j, carry, buf=buf, vb=vb):
            accs = list(carry)
            idxv = jnp.broadcast_to(vb + j, (L,)).astype(jnp.int32)
            for i in range(RPG):
                am = accs[2 * i]
                ai = accs[2 * i + 1]
                v = buf[i, pl.ds(j * L, L)]
                gt = v > am
                accs[2 * i] = jnp.where(gt, v, am)
                accs[2 * i + 1] = jnp.where(gt, idxv, ai)
            return tuple(accs)

        carry = tuple(x for pair in zip(ams, ais) for x in pair)
        carry = lax.fori_loop(0, CW // L, step, carry, unroll=2)
        ams = list(carry[0::2])
        ais = list(carry[1::2])

    mxv = negv
    ixv = zeroi
    for i in range(RPG):
        mv = ams[i]
        mi = ais[i] * L + lane_iota
        for shift in (1, 2, 4, 8):
            pv = _perm(mv, lane_iota ^ shift)
            pi = _perm(mi, lane_iota ^ shift)
            take = (pv > mv) | ((pv == mv) & (pi < mi))
            mv = jnp.where(take, pv, mv)
            mi = jnp.where(take, pi, mi)
        sel = lane_iota == i
        mxv = jnp.where(sel, mv, mxv)
        ixv = jnp.where(sel, mi, ixv)

    mx_v[...] = mxv
    ix_v[...] = ixv
    pltpu.sync_copy(mx_v, mx_hbm.at[wid])
    pltpu.sync_copy(ix_v, ix_hbm.at[wid])


@functools.cache
def _sc_rowmax():
    return functools.partial(
        pl.kernel,
        out_type=(
            jax.ShapeDtypeStruct((NW, L), jnp.float32),
            jax.ShapeDtypeStruct((NW, L), jnp.int32),
        ),
        mesh=plsc.VectorSubcoreMesh(
            core_axis_name="c", subcore_axis_name="s", num_cores=NC, num_subcores=NS
        ),
        scratch_types=[
            pltpu.VMEM((RPG, CW), jnp.float32),
            pltpu.VMEM((RPG, CW), jnp.float32),
            pltpu.VMEM((L,), jnp.float32),
            pltpu.VMEM((L,), jnp.int32),
            pltpu.SemaphoreType.DMA,
            pltpu.SemaphoreType.DMA,
        ],
    )(_sc_body)


def _tc_rowmax_body(x_ref, mx_ref, ix_ref, am_ref, ai_ref):
    i = pl.program_id(0)

    @pl.when(i == 0)
    def _init():
        am_ref[...] = jnp.full((B, 128), NEG_INF, jnp.float32)
        ai_ref[...] = jnp.zeros((B, 128), jnp.int32)

    NJ = TBW // 128

    def fold(masked):
        steps = [
            jnp.broadcast_to(i * NJ + j, (8, 128)).astype(jnp.int32)
            for j in range(NJ)
        ]
        for r in range(B // 8):
            rs = slice(r * 8, (r + 1) * 8)
            am = am_ref[rs, :]
            ai = ai_ref[rs, :]
            for j in range(NJ):
                v = x_ref[rs, j * 128 : (j + 1) * 128]
                if masked:
                    col = (TC_OFF + i) * TBW + j * 128 + jax.lax.broadcasted_iota(
                        jnp.int32, (8, 128), 1
                    )
                    v = jnp.where(col < V, v, NEG_INF)
                gt = v > am
                am = jnp.maximum(am, v)
                ai = jnp.where(gt, steps[j], ai)
            am_ref[rs, :] = am
            ai_ref[rs, :] = ai

    @pl.when(i < TC_STEPS - 1)
    def _hot():
        fold(masked=False)

    @pl.when(i == TC_STEPS - 1)
    def _last():
        fold(masked=True)
        lane = jax.lax.broadcasted_iota(jnp.int32, (B, 128), 1)
        a = am_ref[...]
        cols = ai_ref[...] * 128 + lane + SCW
        m = jnp.max(a, axis=1, keepdims=True)
        cand = jnp.where(a == m, cols, BIG_I)
        mx_ref[...] = m
        ix_ref[...] = jnp.min(cand, axis=1, keepdims=True)


@functools.cache
def _tc_rowmax():
    return pl.pallas_call(
        _tc_rowmax_body,
        grid=(TC_STEPS,),
        in_specs=[pl.BlockSpec((B, TBW), lambda i: (0, TC_OFF + i))],
        out_specs=[
            pl.BlockSpec((B, 1), lambda i: (0, 0)),
            pl.BlockSpec((B, 1), lambda i: (0, 0)),
        ],
        out_shape=(
            jax.ShapeDtypeStruct((B, 1), jnp.float32),
            jax.ShapeDtypeStruct((B, 1), jnp.int32),
        ),
        scratch_shapes=[
            pltpu.VMEM((B, 128), jnp.float32),
            pltpu.VMEM((B, 128), jnp.int32),
        ],
    )


def _finalize_body(mx_ref, ix_ref, tm_ref, ti_ref, t_ref, o_ref):
    tm = jnp.concatenate(
        [tm_ref[...].reshape(G, RPG), jnp.full((G, L - RPG), NEG_INF, jnp.float32)],
        axis=1,
    )
    ti = jnp.concatenate(
        [ti_ref[...].reshape(G, RPG), jnp.zeros((G, L - RPG), jnp.int32)], axis=1
    )
    best_m = jnp.full((G, L), NEG_INF, jnp.float32)
    best_i = jnp.zeros((G, L), jnp.int32)
    for cm, ci in (
        (mx_ref[0:G, :], ix_ref[0:G, :]),
        (mx_ref[G : 2 * G, :], ix_ref[G : 2 * G, :]),
        (tm, ti),
    ):
        take = (cm > best_m) | ((cm == best_m) & (ci < best_i))
        best_m = jnp.where(take, cm, best_m)
        best_i = jnp.where(take, ci, best_i)
    ok = (best_i == t_ref[...]).astype(jnp.float32)
    o_ref[0] = jnp.sum(ok) * (100.0 / B)


_finalize = pl.pallas_call(
    _finalize_body,
    out_shape=jax.ShapeDtypeStruct((1,), jnp.float32),
    in_specs=[pl.BlockSpec(memory_space=pltpu.VMEM)] * 5,
    out_specs=pl.BlockSpec(memory_space=pltpu.SMEM),
)


@jax.jit
def kernel(output, target):
    mx, ix = _sc_rowmax()(output)
    tm, ti = _tc_rowmax()(output)
    tpad = jnp.pad(
        target.reshape(G, RPG), ((0, 0), (0, L - RPG)), constant_values=-1
    )
    return _finalize(mx, ix, tm, ti, tpad)


# speedup vs baseline: 1.9978x; 1.2253x over previous
import functools

import jax
import jax.numpy as jnp
from jax import lax
from jax.experimental import pallas as pl
from jax.experimental.pallas import tpu as pltpu
from jax.experimental.pallas import tpu_sc as plsc

NC = 2
NS = 16
L = 16
NW = NC * NS

B = 128
V = 100000
RPG = 8
G = B // RPG

MW = 20480
SCW = 2 * MW
CW = 5120
NCHK = MW // CW

TBW = 4096
TC_OFF = SCW // TBW
TC_STEPS = (V + TBW - 1) // TBW - TC_OFF

NEG_INF = float("-inf")
BIG_I = 2**30

_GATHER_DNUMS = lax.GatherDimensionNumbers(
    offset_dims=(), collapsed_slice_dims=(0,), start_index_map=(0,)
)


def _perm(x, idx):
    return lax.gather(
        x,
        idx[:, None],
        dimension_numbers=_GATHER_DNUMS,
        slice_sizes=(1,),
        mode=lax.GatherScatterMode.PROMISE_IN_BOUNDS,
    )


def _sc_body(out_hbm, mx_hbm, ix_hbm, bufa, bufb, mx_v, ix_v, sem0, sem1):
    c = lax.axis_index("c")
    s = lax.axis_index("s")
    wid = s * NC + c
    g = wid % G
    h = wid // G
    r0 = g * RPG
    cbase = h * MW

    bufs = (bufa, bufb)
    sems = (sem0, sem1)

    def src(t):
        return out_hbm.at[pl.ds(r0, RPG), pl.ds(cbase + t * CW, CW)]

    pend = [None, None]
    pend[0] = pltpu.async_copy(src(0), bufs[0], sems[0])

    lane_iota = lax.iota(jnp.int32, L)
    negv = jnp.full((L,), NEG_INF, jnp.float32)
    zeroi = jnp.zeros((L,), jnp.int32)
    ams = [negv] * RPG
    ais = [zeroi] * RPG

    for t in range(NCHK):
        if t + 1 < NCHK:
            pend[(t + 1) % 2] = pltpu.async_copy(
                src(t + 1), bufs[(t + 1) % 2], sems[(t + 1) % 2]
            )
        pend[t % 2].wait()
        buf = bufs[t % 2]
        vb = (cbase + t * CW) // L

        def step(j, carry, buf=buf, vb=vb):
            accs = list(carry)
            idxv = jnp.broadcast_to(vb + j, (L,)).astype(jnp.int32)
            for i in range(RPG):
                am = accs[2 * i]
                ai = accs[2 * i + 1]
                v = buf[i, pl.ds(j * L, L)]
                gt = v > am
                accs[2 * i] = jnp.where(gt, v, am)
                accs[2 * i + 1] = jnp.where(gt, idxv, ai)
            return tuple(accs)

        carry = tuple(x for pair in zip(ams, ais) for x in pair)
        carry = lax.fori_loop(0, CW // L, step, carry, unroll=2)
        ams = list(carry[0::2])
        ais = list(carry[1::2])

    mxv = negv
    ixv = zeroi
    for i in range(RPG):
        mv = ams[i]
        mi = ais[i] * L + lane_iota
        for shift in (1, 2, 4, 8):
            pv = _perm(mv, lane_iota ^ shift)
            pi = _perm(mi, lane_iota ^ shift)
            take = (pv > mv) | ((pv == mv) & (pi < mi))
            mv = jnp.where(take, pv, mv)
            mi = jnp.where(take, pi, mi)
        sel = lane_iota == i
        mxv = jnp.where(sel, mv, mxv)
        ixv = jnp.where(sel, mi, ixv)

    mx_v[...] = mxv
    ix_v[...] = ixv
    pltpu.sync_copy(mx_v, mx_hbm.at[wid])
    pltpu.sync_copy(ix_v, ix_hbm.at[wid])


@functools.cache
def _sc_rowmax():
    return functools.partial(
        pl.kernel,
        out_type=(
            jax.ShapeDtypeStruct((NW, L), jnp.float32),
            jax.ShapeDtypeStruct((NW, L), jnp.int32),
        ),
        mesh=plsc.VectorSubcoreMesh(
            core_axis_name="c", subcore_axis_name="s", num_cores=NC, num_subcores=NS
        ),
        scratch_types=[
            pltpu.VMEM((RPG, CW), jnp.float32),
            pltpu.VMEM((RPG, CW), jnp.float32),
            pltpu.VMEM((L,), jnp.float32),
            pltpu.VMEM((L,), jnp.int32),
            pltpu.SemaphoreType.DMA,
            pltpu.SemaphoreType.DMA,
        ],
    )(_sc_body)


def _tc_rowmax_body(x_ref, mx_ref, ix_ref, am_ref, ai_ref):
    i = pl.program_id(0)

    @pl.when(i == 0)
    def _init():
        am_ref[...] = jnp.full((B, 128), NEG_INF, jnp.float32)
        ai_ref[...] = jnp.zeros((B, 128), jnp.int32)

    NJ = TBW // 128

    def fold(masked):
        steps = [
            jnp.broadcast_to(i * NJ + j, (8, 128)).astype(jnp.int32)
            for j in range(NJ)
        ]
        for r in range(B // 8):
            rs = slice(r * 8, (r + 1) * 8)
            am = am_ref[rs, :]
            ai = ai_ref[rs, :]
            for j in range(NJ):
                v = x_ref[rs, j * 128 : (j + 1) * 128]
                if masked:
                    col = (TC_OFF + i) * TBW + j * 128 + jax.lax.broadcasted_iota(
                        jnp.int32, (8, 128), 1
                    )
                    v = jnp.where(col < V, v, NEG_INF)
                gt = v > am
                am = jnp.maximum(am, v)
                ai = jnp.where(gt, steps[j], ai)
            am_ref[rs, :] = am
            ai_ref[rs, :] = ai

    @pl.when(i < TC_STEPS - 1)
    def _hot():
        fold(masked=False)

    @pl.when(i == TC_STEPS - 1)
    def _last():
        fold(masked=True)
        lane = jax.lax.broadcasted_iota(jnp.int32, (B, 128), 1)
        a = am_ref[...]
        cols = ai_ref[...] * 128 + lane + SCW
        m = jnp.max(a, axis=1, keepdims=True)
        cand = jnp.where(a == m, cols, BIG_I)
        mx_ref[...] = m
        ix_ref[...] = jnp.min(cand, axis=1, keepdims=True)


@functools.cache
def _tc_rowmax():
    return pl.pallas_call(
        _tc_rowmax_body,
        grid=(TC_STEPS,),
        in_specs=[pl.BlockSpec((B, TBW), lambda i: (0, TC_OFF + i))],
        out_specs=[
            pl.BlockSpec((B, 1), lambda i: (0, 0)),
            pl.BlockSpec((B, 1), lambda i: (0, 0)),
        ],
        out_shape=(
            jax.ShapeDtypeStruct((B, 1), jnp.float32),
            jax.ShapeDtypeStruct((B, 1), jnp.int32),
        ),
        scratch_shapes=[
            pltpu.VMEM((B, 128), jnp.float32),
            pltpu.VMEM((B, 128), jnp.int32),
        ],
    )


def _finalize_body(mx_ref, ix_ref, tm_ref, ti_ref, t_ref, o_ref):
    tm = jnp.concatenate(
        [tm_ref[...].reshape(G, RPG), jnp.full((G, L - RPG), NEG_INF, jnp.float32)],
        axis=1,
    )
    ti = jnp.concatenate(
        [ti_ref[...].reshape(G, RPG), jnp.zeros((G, L - RPG), jnp.int32)], axis=1
    )
    best_m = jnp.full((G, L), NEG_INF, jnp.float32)
    best_i = jnp.zeros((G, L), jnp.int32)
    for cm, ci in (
        (mx_ref[0:G, :], ix_ref[0:G, :]),
        (mx_ref[G : 2 * G, :], ix_ref[G : 2 * G, :]),
        (tm, ti),
    ):
        take = (cm > best_m) | ((cm == best_m) & (ci < best_i))
        best_m = jnp.where(take, cm, best_m)
        best_i = jnp.where(take, ci, best_i)
    ok = (best_i == t_ref[...]).astype(jnp.float32)
    o_ref[0] = jnp.sum(ok) * (100.0 / B)


_finalize = pl.pallas_call(
    _finalize_body,
    out_shape=jax.ShapeDtypeStruct((1,), jnp.float32),
    in_specs=[pl.BlockSpec(memory_space=pltpu.VMEM)] * 5,
    out_specs=pl.BlockSpec(memory_space=pltpu.SMEM),
)


@jax.jit
def kernel(output, target):
    mx, ix = _sc_rowmax()(output)
    tm, ti = _tc_rowmax()(output)
    tpad = jnp.pad(
        target.reshape(G, RPG), ((0, 0), (0, L - RPG)), constant_values=-1
    )
    return _finalize(mx, ix, tm, ti, tpad)


# speedup vs baseline: 3.7417x; 1.8729x over previous
import functools

import jax
import jax.numpy as jnp
from jax import lax
from jax.experimental import pallas as pl
from jax.experimental.pallas import tpu as pltpu
from jax.experimental.pallas import tpu_sc as plsc

NC = 2
NS = 16
L = 16
NW = NC * NS

B = 128
V = 100000
RG = B // L

SCC = 40960
CLS_W = SCC // NW
CCH = 320
NCHK = CLS_W // CCH

TBW = 2048
TC_OFF = SCC // TBW
TC_STEPS = (V + TBW - 1) // TBW - TC_OFF

NEG_INF = float("-inf")
BIG_I = 2**30


def _sc_body(xt_hbm, mx_hbm, ix_hbm, bufa, bufb, mx_v, ix_v, sem0, sem1):
    c = lax.axis_index("c")
    s = lax.axis_index("s")
    wid = s * NC + c
    cls0 = wid * CLS_W

    bufs = (bufa, bufb)
    sems = (sem0, sem1)

    def src(t):
        return xt_hbm.at[pl.ds(cls0 + t * CCH, CCH), :]

    pend = [None, None]
    pend[0] = pltpu.async_copy(src(0), bufs[0], sems[0])

    negv = jnp.full((L,), NEG_INF, jnp.float32)
    zeroi = jnp.zeros((L,), jnp.int32)
    ams = [negv] * RG
    ais = [zeroi] * RG

    for t in range(NCHK):
        if t + 1 < NCHK:
            pend[(t + 1) % 2] = pltpu.async_copy(
                src(t + 1), bufs[(t + 1) % 2], sems[(t + 1) % 2]
            )
        pend[t % 2].wait()
        buf = bufs[t % 2]
        cb = cls0 + t * CCH

        def step(j, carry, buf=buf, cb=cb):
            accs = list(carry)
            idxv = jnp.broadcast_to(cb + j, (L,)).astype(jnp.int32)
            for r in range(RG):
                am = accs[2 * r]
                ai = accs[2 * r + 1]
                v = buf[j, pl.ds(r * L, L)]
                gt = v > am
                accs[2 * r] = jnp.where(gt, v, am)
                accs[2 * r + 1] = jnp.where(gt, idxv, ai)
            return tuple(accs)

        carry = tuple(x for pair in zip(ams, ais) for x in pair)
        carry = lax.fori_loop(0, CCH, step, carry, unroll=2)
        ams = list(carry[0::2])
        ais = list(carry[1::2])

    for r in range(RG):
        mx_v[pl.ds(r * L, L)] = ams[r]
        ix_v[pl.ds(r * L, L)] = ais[r]
    pltpu.sync_copy(mx_v, mx_hbm.at[wid])
    pltpu.sync_copy(ix_v, ix_hbm.at[wid])


@functools.cache
def _sc_rowmax():
    return functools.partial(
        pl.kernel,
        out_type=(
            jax.ShapeDtypeStruct((NW, B), jnp.float32),
            jax.ShapeDtypeStruct((NW, B), jnp.int32),
        ),
        mesh=plsc.VectorSubcoreMesh(
            core_axis_name="c", subcore_axis_name="s", num_cores=NC, num_subcores=NS
        ),
        scratch_types=[
            pltpu.VMEM((CCH, B), jnp.float32),
            pltpu.VMEM((CCH, B), jnp.float32),
            pltpu.VMEM((B,), jnp.float32),
            pltpu.VMEM((B,), jnp.int32),
            pltpu.SemaphoreType.DMA,
            pltpu.SemaphoreType.DMA,
        ],
    )(_sc_body)


def _tc_rowmax_body(x_ref, mx_ref, ix_ref, am_ref, ai_ref):
    i = pl.program_id(0)

    @pl.when(i == 0)
    def _init():
        am_ref[...] = jnp.full((8, B), NEG_INF, jnp.float32)
        ai_ref[...] = jnp.zeros((8, B), jnp.int32)

    NJ = TBW // 8

    def fold(masked):
        am = am_ref[...]
        ai = ai_ref[...]
        for j in range(NJ):
            v = x_ref[j * 8 : (j + 1) * 8, :]
            if masked:
                cls = (TC_OFF + i) * TBW + j * 8 + jax.lax.broadcasted_iota(
                    jnp.int32, (8, B), 0
                )
                v = jnp.where(cls < V, v, NEG_INF)
            octet = jnp.broadcast_to(i * NJ + j, (8, B)).astype(jnp.int32)
            gt = v > am
            am = jnp.maximum(am, v)
            ai = jnp.where(gt, octet, ai)
        am_ref[...] = am
        ai_ref[...] = ai

    @pl.when(i < TC_STEPS - 1)
    def _hot():
        fold(masked=False)

    @pl.when(i == TC_STEPS - 1)
    def _last():
        fold(masked=True)
        a = am_ref[...]
        sub = jax.lax.broadcasted_iota(jnp.int32, (8, B), 0)
        cls = (ai_ref[...] * 8 + sub) + SCC
        m = jnp.max(a, axis=0, keepdims=True)
        cand = jnp.where(a == m, cls, BIG_I)
        mx_ref[...] = m
        ix_ref[...] = jnp.min(cand, axis=0, keepdims=True)


@functools.cache
def _tc_rowmax():
    return pl.pallas_call(
        _tc_rowmax_body,
        grid=(TC_STEPS,),
        in_specs=[pl.BlockSpec((TBW, B), lambda i: (TC_OFF + i, 0))],
        out_specs=[
            pl.BlockSpec((1, B), lambda i: (0, 0)),
            pl.BlockSpec((1, B), lambda i: (0, 0)),
        ],
        out_shape=(
            jax.ShapeDtypeStruct((1, B), jnp.float32),
            jax.ShapeDtypeStruct((1, B), jnp.int32),
        ),
        scratch_shapes=[
            pltpu.VMEM((8, B), jnp.float32),
            pltpu.VMEM((8, B), jnp.int32),
        ],
    )


def _finalize_body(scm_ref, sci_ref, tcm_ref, tci_ref, t_ref, o_ref):
    scm = scm_ref[...]
    sci = sci_ref[...]
    tcm = tcm_ref[...]
    tci = tci_ref[...]
    bm = jnp.maximum(jnp.max(scm, axis=0, keepdims=True), tcm)
    c_sc = jnp.min(jnp.where(scm == bm, sci, BIG_I), axis=0, keepdims=True)
    c_tc = jnp.where(tcm == bm, tci, BIG_I)
    best = jnp.minimum(c_sc, c_tc)
    ok = (best == t_ref[...].reshape(1, B)).astype(jnp.float32)
    o_ref[0] = jnp.sum(ok) * (100.0 / B)


_finalize = pl.pallas_call(
    _finalize_body,
    out_shape=jax.ShapeDtypeStruct((1,), jnp.float32),
    in_specs=[pl.BlockSpec(memory_space=pltpu.VMEM)] * 5,
    out_specs=pl.BlockSpec(memory_space=pltpu.SMEM),
)


@jax.jit
def kernel(output, target):
    xt = output.T
    mx, ix = _sc_rowmax()(xt)
    tm, ti = _tc_rowmax()(xt)
    return _finalize(mx, ix, tm, ti, target)


# speedup vs baseline: 4.2669x; 1.1404x over previous
import functools

import jax
import jax.numpy as jnp
from jax import lax
from jax.experimental import pallas as pl
from jax.experimental.pallas import tpu as pltpu
from jax.experimental.pallas import tpu_sc as plsc

NC = 2
NS = 16
L = 16
NW = NC * NS

B = 128
V = 100000
RG = B // L

SCC = 40960
CLS_W = SCC // NW
CCH = 320
NCHK = CLS_W // CCH

TBW = 4096
TC_OFF = SCC // TBW
TC_STEPS = (V + TBW - 1) // TBW - TC_OFF

NEG_INF = float("-inf")
BIG_I = 2**30


def _sc_body(xt_hbm, mx_hbm, ix_hbm, bufa, bufb, mx_v, ix_v, sem0, sem1):
    c = lax.axis_index("c")
    s = lax.axis_index("s")
    wid = s * NC + c
    cls0 = wid * CLS_W

    bufs = (bufa, bufb)
    sems = (sem0, sem1)

    def src(t):
        return xt_hbm.at[pl.ds(cls0 + t * CCH, CCH), :]

    pend = [None, None]
    pend[0] = pltpu.async_copy(src(0), bufs[0], sems[0])

    negv = jnp.full((L,), NEG_INF, jnp.float32)
    zeroi = jnp.zeros((L,), jnp.int32)
    ams = [negv] * RG
    ais = [zeroi] * RG

    for t in range(NCHK):
        if t + 1 < NCHK:
            pend[(t + 1) % 2] = pltpu.async_copy(
                src(t + 1), bufs[(t + 1) % 2], sems[(t + 1) % 2]
            )
        pend[t % 2].wait()
        buf = bufs[t % 2]
        cb = cls0 + t * CCH

        def step(j, carry, buf=buf, cb=cb):
            accs = list(carry)
            idxv = jnp.broadcast_to(cb + j, (L,)).astype(jnp.int32)
            for r in range(RG):
                am = accs[2 * r]
                ai = accs[2 * r + 1]
                v = buf[j, pl.ds(r * L, L)]
                gt = v > am
                accs[2 * r] = jnp.where(gt, v, am)
                accs[2 * r + 1] = jnp.where(gt, idxv, ai)
            return tuple(accs)

        carry = tuple(x for pair in zip(ams, ais) for x in pair)
        carry = lax.fori_loop(0, CCH, step, carry, unroll=2)
        ams = list(carry[0::2])
        ais = list(carry[1::2])

    for r in range(RG):
        mx_v[pl.ds(r * L, L)] = ams[r]
        ix_v[pl.ds(r * L, L)] = ais[r]
    pltpu.sync_copy(mx_v, mx_hbm.at[wid])
    pltpu.sync_copy(ix_v, ix_hbm.at[wid])


@functools.cache
def _sc_rowmax():
    return functools.partial(
        pl.kernel,
        out_type=(
            jax.ShapeDtypeStruct((NW, B), jnp.float32),
            jax.ShapeDtypeStruct((NW, B), jnp.int32),
        ),
        mesh=plsc.VectorSubcoreMesh(
            core_axis_name="c", subcore_axis_name="s", num_cores=NC, num_subcores=NS
        ),
        scratch_types=[
            pltpu.VMEM((CCH, B), jnp.float32),
            pltpu.VMEM((CCH, B), jnp.float32),
            pltpu.VMEM((B,), jnp.float32),
            pltpu.VMEM((B,), jnp.int32),
            pltpu.SemaphoreType.DMA,
            pltpu.SemaphoreType.DMA,
        ],
    )(_sc_body)


def _tc_rowmax_body(x_ref, mx_ref, ix_ref, am_ref, ai_ref):
    i = pl.program_id(0)

    @pl.when(i == 0)
    def _init():
        am_ref[...] = jnp.full((8, B), NEG_INF, jnp.float32)
        ai_ref[...] = jnp.zeros((8, B), jnp.int32)

    NJ = TBW // 8

    def fold(masked):
        am = am_ref[...]
        ai = ai_ref[...]
        octet = jnp.broadcast_to(i * NJ, (8, B)).astype(jnp.int32)
        one = jnp.ones((8, B), jnp.int32)
        for j in range(NJ):
            v = x_ref[j * 8 : (j + 1) * 8, :]
            if masked:
                cls = (TC_OFF + i) * TBW + j * 8 + jax.lax.broadcasted_iota(
                    jnp.int32, (8, B), 0
                )
                v = jnp.where(cls < V, v, NEG_INF)
            gt = v > am
            am = jnp.maximum(am, v)
            ai = jnp.where(gt, octet, ai)
            octet = octet + one
        am_ref[...] = am
        ai_ref[...] = ai

    @pl.when(i < TC_STEPS - 1)
    def _hot():
        fold(masked=False)

    @pl.when(i == TC_STEPS - 1)
    def _last():
        fold(masked=True)
        a = am_ref[...]
        sub = jax.lax.broadcasted_iota(jnp.int32, (8, B), 0)
        cls = (ai_ref[...] * 8 + sub) + SCC
        m = jnp.max(a, axis=0, keepdims=True)
        cand = jnp.where(a == m, cls, BIG_I)
        mx_ref[...] = m
        ix_ref[...] = jnp.min(cand, axis=0, keepdims=True)


@functools.cache
def _tc_rowmax():
    return pl.pallas_call(
        _tc_rowmax_body,
        grid=(TC_STEPS,),
        in_specs=[pl.BlockSpec((TBW, B), lambda i: (TC_OFF + i, 0))],
        out_specs=[
            pl.BlockSpec((1, B), lambda i: (0, 0)),
            pl.BlockSpec((1, B), lambda i: (0, 0)),
        ],
        out_shape=(
            jax.ShapeDtypeStruct((1, B), jnp.float32),
            jax.ShapeDtypeStruct((1, B), jnp.int32),
        ),
        scratch_shapes=[
            pltpu.VMEM((8, B), jnp.float32),
            pltpu.VMEM((8, B), jnp.int32),
        ],
    )


def _finalize_body(scm_ref, sci_ref, tcm_ref, tci_ref, t_ref, o_ref):
    scm = scm_ref[...]
    sci = sci_ref[...]
    tcm = tcm_ref[...]
    tci = tci_ref[...]
    bm = jnp.maximum(jnp.max(scm, axis=0, keepdims=True), tcm)
    c_sc = jnp.min(jnp.where(scm == bm, sci, BIG_I), axis=0, keepdims=True)
    c_tc = jnp.where(tcm == bm, tci, BIG_I)
    best = jnp.minimum(c_sc, c_tc)
    ok = (best == t_ref[...].reshape(1, B)).astype(jnp.float32)
    o_ref[0] = jnp.sum(ok) * (100.0 / B)


_finalize = pl.pallas_call(
    _finalize_body,
    out_shape=jax.ShapeDtypeStruct((1,), jnp.float32),
    in_specs=[pl.BlockSpec(memory_space=pltpu.VMEM)] * 5,
    out_specs=pl.BlockSpec(memory_space=pltpu.SMEM),
)


@jax.jit
def kernel(output, target):
    xt = output.T
    mx, ix = _sc_rowmax()(xt)
    tm, ti = _tc_rowmax()(xt)
    return _finalize(mx, ix, tm, ti, target)
